# baseline (speedup 1.0000x reference)
import functools
import math

import numpy as np
import jax
import jax.numpy as jnp
from jax import lax
from jax.experimental import pallas as pl
from jax.experimental.pallas import tpu as pltpu
from jax.experimental.pallas import tpu_sc as plsc

PS = 16
MASK_RATIO = 0.75
MIN_MASK = 4
MAX_MASK = 48

B, C, H, W = 4, 96, 384, 384
NPH, NPW = H // PS, W // PS
TOTAL = NPH * NPW

_num_masked = max(MIN_MASK, min(int(TOTAL * MASK_RATIO), MAX_MASK))
_bs = int(math.sqrt(_num_masked))
_ch, _cw = NPH // 2, NPW // 2
_MASK_IDS = [i * NPW + j
             for i in range(max(0, _ch - _bs // 2), min(NPH, _ch + _bs // 2))
             for j in range(max(0, _cw - _bs // 2), min(NPW, _cw + _bs // 2))]
_mask_row = np.zeros(TOTAL, dtype=bool)
_mask_row[_MASK_IDS] = True
_KEEP = np.nonzero(~_mask_row)[0]
NKEEP = len(_KEEP)

_mi = np.asarray(_MASK_IDS) // NPW
_mj = np.asarray(_MASK_IDS) % NPW
R0, R1 = int(_mi.min()) * PS, (int(_mi.max()) + 1) * PS
C0, C1 = int(_mj.min()) * PS, (int(_mj.max()) + 1) * PS

_MASK_CONST = np.tile(_mask_row[None, :], (B, 1))

NTR = (NKEEP + 7) // 8
KPAD = NTR * 8
NUNITS = B * NTR * 2
NW = 32
UPW = NUNITS // NW
ROWS_PER_UNIT = 768
NROWS_OUT = NUNITS * ROWS_PER_UNIT
NROWS_TAB = B * C * NPW * (H // 8)


def _gather_index() -> np.ndarray:
    u = np.arange(NUNITS)
    bu = u // (NTR * 2)
    k0 = (u % (NTR * 2)) * 4
    pos = np.arange(ROWS_PER_UNIT)
    k = k0[:, None] + pos[None, :] // 192
    tc = np.broadcast_to(pos[None, :] % 192, k.shape)
    kk = np.minimum(k, NKEEP - 1)
    p = _KEEP[kk]
    i = p // NPW
    j = p % NPW
    c = tc // 2
    hh = 2 * i + tc % 2
    r = ((bu[:, None] * C + c) * NPW + j) * 48 + hh
    r = np.where(k > NKEEP - 1, 0, r).astype(np.int32)
    r = r.reshape(NUNITS, 6, 128)
    rp = np.zeros((NW, UPW, 8, 128), np.int32)
    rp[:, :, :6, :] = r.reshape(NW, UPW, 6, 128)
    return rp


_IDX4 = _gather_index()


def _sc_gather(xt_rows, idx4):
    mesh = plsc.VectorSubcoreMesh(core_axis_name="c", subcore_axis_name="s")

    @functools.partial(
        pl.kernel,
        mesh=mesh,
        compiler_params=pltpu.CompilerParams(use_tc_tiling_on_sc=True),
        out_type=jax.ShapeDtypeStruct((NROWS_OUT, 128), jnp.float32),
        scratch_types=[
            pltpu.VMEM((8, 128), jnp.int32),
            pltpu.VMEM((ROWS_PER_UNIT, 128), jnp.float32),
            pltpu.SemaphoreType.DMA,
        ],
    )
    def k(xt_hbm, idx_hbm, out_hbm, idx_v, buf_v, sem):
        wid = lax.axis_index("s") * 2 + lax.axis_index("c")

        def unit(t, carry):
            pltpu.sync_copy(idx_hbm.at[wid, t], idx_v)
            cps = [
                pltpu.async_copy(xt_hbm.at[idx_v.at[d]],
                                 buf_v.at[pl.ds(d * 128, 128)], sem)
                for d in range(6)
            ]
            for cp in cps:
                cp.wait()
            pltpu.sync_copy(
                buf_v,
                out_hbm.at[pl.ds((wid * UPW + t) * ROWS_PER_UNIT,
                                 ROWS_PER_UNIT)])
            return carry

        lax.fori_loop(0, UPW, unit, 0)

    return k(xt_rows, idx4)


def _tc_masked_copy(x3):
    def body(in_ref, out_ref):
        r = lax.broadcasted_iota(jnp.int32, (H, W), 0)
        c = lax.broadcasted_iota(jnp.int32, (H, W), 1)
        inside = (r >= R0) & (r < R1) & (c >= C0) & (c < C1)
        out_ref[0] = jnp.where(inside, 0.0, in_ref[0])

    return pl.pallas_call(
        body,
        grid=(B * C,),
        in_specs=[pl.BlockSpec((1, H, W), lambda g: (g, 0, 0))],
        out_specs=pl.BlockSpec((1, H, W), lambda g: (g, 0, 0)),
        out_shape=jax.ShapeDtypeStruct((B * C, H, W), jnp.float32),
    )(x3)


def kernel(x):
    x3 = x.reshape(B * C, H, W)
    masked_input = _tc_masked_copy(x3).reshape(B, C, H, W)

    xt = x.reshape(B, C, H // 8, 8, NPW, PS).transpose(0, 1, 4, 2, 3, 5)
    xt_rows = xt.reshape(NROWS_TAB, 128)
    out2d = _sc_gather(xt_rows, jnp.asarray(_IDX4))
    unmasked_patches = out2d.reshape(B, KPAD, C * PS * PS)[:, :NKEEP]

    mask = jnp.asarray(_MASK_CONST)
    return (masked_input, mask, unmasked_patches)


# speedup vs baseline: 2.1099x; 2.1099x over previous
import functools
import math

import numpy as np
import jax
import jax.numpy as jnp
from jax import lax
from jax.experimental import pallas as pl
from jax.experimental.pallas import tpu as pltpu
from jax.experimental.pallas import tpu_sc as plsc

PS = 16
MASK_RATIO = 0.75
MIN_MASK = 4
MAX_MASK = 48

B, C, H, W = 4, 96, 384, 384
NPH, NPW = H // PS, W // PS
TOTAL = NPH * NPW

_num_masked = max(MIN_MASK, min(int(TOTAL * MASK_RATIO), MAX_MASK))
_bs = int(math.sqrt(_num_masked))
_ch, _cw = NPH // 2, NPW // 2
_MASK_IDS = [i * NPW + j
             for i in range(max(0, _ch - _bs // 2), min(NPH, _ch + _bs // 2))
             for j in range(max(0, _cw - _bs // 2), min(NPW, _cw + _bs // 2))]
_mask_row = np.zeros(TOTAL, dtype=bool)
_mask_row[_MASK_IDS] = True
_KEEP = np.nonzero(~_mask_row)[0]
NKEEP = len(_KEEP)

_mi = np.asarray(_MASK_IDS) // NPW
_mj = np.asarray(_MASK_IDS) % NPW
R0, R1 = int(_mi.min()) * PS, (int(_mi.max()) + 1) * PS
C0, C1 = int(_mj.min()) * PS, (int(_mj.max()) + 1) * PS

_MASK_CONST = np.tile(_mask_row[None, :], (B, 1))

KPAD = 544
NROWS_OUT = B * KPAD * C * PS
NROWS_TAB = B * C * H * NPW
NW = 32
RPW = NROWS_OUT // NW
DMAS_PER_STEP = 12
ROWS_PER_STEP = DMAS_PER_STEP * 128
STEPS = RPW // ROWS_PER_STEP
assert RPW % ROWS_PER_STEP == 0


def _gather_index() -> np.ndarray:
    b = np.arange(B)[:, None, None, None]
    k = np.arange(KPAD)[None, :, None, None]
    c = np.arange(C)[None, None, :, None]
    pi = np.arange(PS)[None, None, None, :]
    p = _KEEP[np.minimum(k, NKEEP - 1)]
    i = p // NPW
    j = p % NPW
    r = ((b * C + c) * H + i * PS + pi) * NPW + j
    r = np.where(k > NKEEP - 1, 0, r).astype(np.int32)
    return r.reshape(NW, STEPS, DMAS_PER_STEP, 128)


_IDX4 = _gather_index()


def _sc_gather(x_rows, idx4):
    mesh = plsc.VectorSubcoreMesh(core_axis_name="c", subcore_axis_name="s")

    @functools.partial(
        pl.kernel,
        mesh=mesh,
        compiler_params=pltpu.CompilerParams(use_tc_tiling_on_sc=False),
        out_type=jax.ShapeDtypeStruct((NROWS_OUT, PS), jnp.float32),
        scratch_types=[
            pltpu.VMEM((DMAS_PER_STEP, 128), jnp.int32),
            pltpu.VMEM((ROWS_PER_STEP, PS), jnp.float32),
            pltpu.SemaphoreType.DMA,
        ],
    )
    def k(x_hbm, idx_hbm, out_hbm, idx_v, rows_v, sem):
        wid = lax.axis_index("s") * 2 + lax.axis_index("c")
        base = wid * RPW

        def step(t, carry):
            pltpu.sync_copy(idx_hbm.at[wid, t], idx_v)
            cps = [
                pltpu.async_copy(x_hbm.at[idx_v.at[d]],
                                 rows_v.at[pl.ds(d * 128, 128)], sem)
                for d in range(DMAS_PER_STEP)
            ]
            for cp in cps:
                cp.wait()
            pltpu.sync_copy(
                rows_v,
                out_hbm.at[pl.ds(base + t * ROWS_PER_STEP, ROWS_PER_STEP)])
            return carry

        lax.fori_loop(0, STEPS, step, 0)

    return k(x_rows, idx4)


def _tc_masked_copy(x3):
    def body(in_ref, out_ref):
        r = lax.broadcasted_iota(jnp.int32, (H, W), 0)
        c = lax.broadcasted_iota(jnp.int32, (H, W), 1)
        inside = (r >= R0) & (r < R1) & (c >= C0) & (c < C1)
        out_ref[0] = jnp.where(inside, 0.0, in_ref[0])

    return pl.pallas_call(
        body,
        grid=(B * C,),
        in_specs=[pl.BlockSpec((1, H, W), lambda g: (g, 0, 0))],
        out_specs=pl.BlockSpec((1, H, W), lambda g: (g, 0, 0)),
        out_shape=jax.ShapeDtypeStruct((B * C, H, W), jnp.float32),
    )(x3)


def kernel(x):
    x3 = x.reshape(B * C, H, W)
    masked_input = _tc_masked_copy(x3).reshape(B, C, H, W)

    x_rows = x.reshape(NROWS_TAB, PS)
    out2d = _sc_gather(x_rows, jnp.asarray(_IDX4))
    unmasked_patches = out2d.reshape(B, KPAD, C * PS * PS)[:, :NKEEP]

    mask = jnp.asarray(_MASK_CONST)
    return (masked_input, mask, unmasked_patches)


# speedup vs baseline: 2.2588x; 1.0706x over previous
import functools
import math

import numpy as np
import jax
import jax.numpy as jnp
from jax import lax
from jax.experimental import pallas as pl
from jax.experimental.pallas import tpu as pltpu
from jax.experimental.pallas import tpu_sc as plsc

PS = 16
MASK_RATIO = 0.75
MIN_MASK = 4
MAX_MASK = 48

B, C, H, W = 4, 96, 384, 384
NPH, NPW = H // PS, W // PS
TOTAL = NPH * NPW

_num_masked = max(MIN_MASK, min(int(TOTAL * MASK_RATIO), MAX_MASK))
_bs = int(math.sqrt(_num_masked))
_ch, _cw = NPH // 2, NPW // 2
_MASK_IDS = [i * NPW + j
             for i in range(max(0, _ch - _bs // 2), min(NPH, _ch + _bs // 2))
             for j in range(max(0, _cw - _bs // 2), min(NPW, _cw + _bs // 2))]
_mask_row = np.zeros(TOTAL, dtype=bool)
_mask_row[_MASK_IDS] = True
_KEEP = np.nonzero(~_mask_row)[0]
NKEEP = len(_KEEP)

_mi = np.asarray(_MASK_IDS) // NPW
_mj = np.asarray(_MASK_IDS) % NPW
R0, R1 = int(_mi.min()) * PS, (int(_mi.max()) + 1) * PS
C0, C1 = int(_mj.min()) * PS, (int(_mj.max()) + 1) * PS

_MASK_CONST = np.tile(_mask_row[None, :], (B, 1))

KPAD = 544
NROWS_OUT = B * KPAD * C * PS
NROWS_TAB = B * C * H * NPW
NW = 32
RPW = NROWS_OUT // NW
DMAS_PER_STEP = 12
ROWS_PER_STEP = DMAS_PER_STEP * 128
STEPS = RPW // ROWS_PER_STEP
assert RPW % ROWS_PER_STEP == 0


def _gather_index() -> np.ndarray:
    b = np.arange(B)[:, None, None, None]
    k = np.arange(KPAD)[None, :, None, None]
    c = np.arange(C)[None, None, :, None]
    pi = np.arange(PS)[None, None, None, :]
    p = _KEEP[np.minimum(k, NKEEP - 1)]
    i = p // NPW
    j = p % NPW
    r = ((b * C + c) * H + i * PS + pi) * NPW + j
    r = np.where(k > NKEEP - 1, 0, r).astype(np.int32)
    return r.reshape(NW, STEPS, DMAS_PER_STEP, 128)


_IDX4 = _gather_index()


def _sc_gather(x_rows, idx4):
    mesh = plsc.VectorSubcoreMesh(core_axis_name="c", subcore_axis_name="s")

    @functools.partial(
        pl.kernel,
        mesh=mesh,
        compiler_params=pltpu.CompilerParams(use_tc_tiling_on_sc=False),
        out_type=jax.ShapeDtypeStruct((NROWS_OUT, PS), jnp.float32),
        scratch_types=[
            pltpu.VMEM((DMAS_PER_STEP, 128), jnp.int32),
            pltpu.VMEM((ROWS_PER_STEP, PS), jnp.float32),
            pltpu.SemaphoreType.DMA,
        ],
    )
    def k(x_hbm, idx_hbm, out_hbm, idx_v, rows_v, sem):
        wid = lax.axis_index("s") * 2 + lax.axis_index("c")
        base = wid * RPW

        def step(t, carry):
            pltpu.sync_copy(idx_hbm.at[wid, t], idx_v)
            cps = [
                pltpu.async_copy(x_hbm.at[idx_v.at[d]],
                                 rows_v.at[pl.ds(d * 128, 128)], sem)
                for d in range(DMAS_PER_STEP)
            ]
            for cp in cps:
                cp.wait()
            pltpu.sync_copy(
                rows_v,
                out_hbm.at[pl.ds(base + t * ROWS_PER_STEP, ROWS_PER_STEP)])
            return carry

        lax.fori_loop(0, STEPS, step, 0)

    return k(x_rows, idx4)


_MROWS = 4096


def _tc_masked_copy(x2):
    def body(in_ref, out_ref):
        g = pl.program_id(0)
        h = (lax.broadcasted_iota(jnp.int32, (_MROWS, W), 0) + g * _MROWS) % H
        c = lax.broadcasted_iota(jnp.int32, (_MROWS, W), 1)
        inside = (h >= R0) & (h < R1) & (c >= C0) & (c < C1)
        out_ref[...] = jnp.where(inside, 0.0, in_ref[...])

    return pl.pallas_call(
        body,
        grid=(B * C * H // _MROWS,),
        in_specs=[pl.BlockSpec((_MROWS, W), lambda g: (g, 0))],
        out_specs=pl.BlockSpec((_MROWS, W), lambda g: (g, 0)),
        out_shape=jax.ShapeDtypeStruct((B * C * H, W), jnp.float32),
    )(x2)


def kernel(x):
    x_rows = x.reshape(NROWS_TAB, PS)
    out2d = _sc_gather(x_rows, jnp.asarray(_IDX4))

    x2 = x.reshape(B * C * H, W)
    masked_input = _tc_masked_copy(x2).reshape(B, C, H, W)

    unmasked_patches = out2d.reshape(B, KPAD, C * PS * PS)[:, :NKEEP]

    mask = jnp.asarray(_MASK_CONST)
    return (masked_input, mask, unmasked_patches)
